# baseline (speedup 1.0000x reference)
import functools

import jax
import jax.numpy as jnp
from jax.experimental import pallas as pl
from jax.experimental.pallas import tpu as pltpu

F32 = jnp.float32
BF16 = jnp.bfloat16

_NN = (((1,), (0,)), ((), ()))
_TT = (((0,), (0,)), ((), ()))


def _hi_lo_concat(x, axis=1):
    hi = x.astype(BF16)
    lo = (x - hi.astype(F32)).astype(BF16)
    return jnp.concatenate([hi, lo], axis=axis)


def _sum_halves(y, axis=1):
    w = y.shape[axis] // 2
    if axis == 0:
        return y[:w] + y[w:]
    return y[:, :w] + y[:, w:]


def _xw_body(x_ref, w_ref, o_ref):
    xw = jnp.dot(x_ref[...], w_ref[...], preferred_element_type=F32)
    o_ref[...] = _hi_lo_concat(xw)


def _xw(x, w, bm):
    n, k = x.shape
    h = w.shape[1]
    return pl.pallas_call(
        _xw_body,
        grid=(n // bm,),
        in_specs=[
            pl.BlockSpec((bm, k), lambda i: (i, 0)),
            pl.BlockSpec((k, h), lambda i: (0, 0)),
        ],
        out_specs=pl.BlockSpec((bm, 2 * h), lambda i: (i, 0)),
        out_shape=jax.ShapeDtypeStruct((n, 2 * h), BF16),
    )(x, w)


def _h_body(adj_ref, xw1_ref, xw2_ref, h1_ref, h2_ref, acc1, *, ni):
    i = pl.program_id(0)
    t = adj_ref[...].astype(BF16)
    h2 = jnp.maximum(
        _sum_halves(jax.lax.dot_general(t, xw2_ref[...], _NN,
                                        preferred_element_type=F32)), 0.0)
    h2_ref[...] = _hi_lo_concat(h2)
    c1t = jax.lax.dot_general(xw1_ref[...], t, _TT,
                              preferred_element_type=F32)

    @pl.when(i == 0)
    def _():
        acc1[...] = c1t

    @pl.when(i != 0)
    def _():
        acc1[...] += c1t

    @pl.when(i == ni - 1)
    def _():
        h1t = jnp.maximum(_sum_halves(acc1[...], axis=0), 0.0)
        h1_ref[...] = _hi_lo_concat(h1t.T)


def _propagate_in(adj, xw1, xw2, b):
    n1, n2 = adj.shape
    h2w = xw1.shape[1]
    ni = n1 // b
    return pl.pallas_call(
        functools.partial(_h_body, ni=ni),
        grid=(ni,),
        in_specs=[
            pl.BlockSpec((b, n2), lambda i: (i, 0)),
            pl.BlockSpec((b, h2w), lambda i: (i, 0)),
            pl.BlockSpec((n2, h2w), lambda i: (0, 0)),
        ],
        out_specs=[
            pl.BlockSpec((n2, h2w), lambda i: (0, 0)),
            pl.BlockSpec((b, h2w), lambda i: (i, 0)),
        ],
        out_shape=[
            jax.ShapeDtypeStruct((n2, h2w), BF16),
            jax.ShapeDtypeStruct((n1, h2w), BF16),
        ],
        scratch_shapes=[pltpu.VMEM((h2w, n2), F32)],
    )(adj, xw1, xw2)


def _z_body(adj_ref, h1_ref, h2_ref, wm1_ref, wl1_ref, wm2_ref, wl2_ref,
            n1_ref, n2_ref, z1_ref, z2_ref, acc2, *, ni):
    i = pl.program_id(0)
    t = adj_ref[...].astype(BF16)
    ah1 = _sum_halves(jax.lax.dot_general(t, h1_ref[...], _NN,
                                          preferred_element_type=F32))
    mean1 = jnp.dot(ah1, wm1_ref[...], preferred_element_type=F32)
    logstd1 = jnp.dot(ah1, wl1_ref[...], preferred_element_type=F32)
    z1_ref[...] = n1_ref[...] * jnp.exp(logstd1) + mean1

    c2t = jax.lax.dot_general(h2_ref[...], t, _TT,
                              preferred_element_type=F32)

    @pl.when(i == 0)
    def _():
        acc2[...] = c2t

    @pl.when(i != 0)
    def _():
        acc2[...] += c2t

    @pl.when(i == ni - 1)
    def _():
        ah2t = _sum_halves(acc2[...], axis=0)
        mean2t = jax.lax.dot_general(wm2_ref[...], ah2t, _TT,
                                     preferred_element_type=F32)
        logstd2t = jax.lax.dot_general(wl2_ref[...], ah2t, _TT,
                                       preferred_element_type=F32)
        z2_ref[...] = (n2_ref[...].T * jnp.exp(logstd2t) + mean2t).T


def _propagate_out(adj, h1, h2, wm1, wl1, wm2, wl2, noise1, noise2, b):
    n1, n2 = adj.shape
    h2w = h1.shape[1]
    hz = wm1.shape[1]
    ni = n1 // b
    full = lambda a: pl.BlockSpec(a.shape, lambda i: tuple(0 for _ in a.shape))
    return pl.pallas_call(
        functools.partial(_z_body, ni=ni),
        grid=(ni,),
        in_specs=[
            pl.BlockSpec((b, n2), lambda i: (i, 0)),
            full(h1),
            pl.BlockSpec((b, h2w), lambda i: (i, 0)),
            full(wm1), full(wl1), full(wm2), full(wl2),
            pl.BlockSpec((b, hz), lambda i: (i, 0)),
            full(noise2),
        ],
        out_specs=[
            pl.BlockSpec((b, hz), lambda i: (i, 0)),
            pl.BlockSpec((n2, hz), lambda i: (0, 0)),
        ],
        out_shape=[
            jax.ShapeDtypeStruct((n1, hz), F32),
            jax.ShapeDtypeStruct((n2, hz), F32),
        ],
        scratch_shapes=[pltpu.VMEM((h2w, n2), F32)],
    )(adj, h1, h2, wm1, wl1, wm2, wl2, noise1, noise2)


def _dec_body(z1_ref, z2_ref, a_ref):
    logits = jax.lax.dot_general(z1_ref[...], z2_ref[...],
                                 (((1,), (1,)), ((), ())),
                                 preferred_element_type=F32)
    a_ref[...] = jax.nn.sigmoid(logits)


def _decode(z1, z2, bm):
    n1, hz = z1.shape
    n2 = z2.shape[0]
    return pl.pallas_call(
        _dec_body,
        grid=(n1 // bm,),
        in_specs=[
            pl.BlockSpec((bm, hz), lambda i: (i, 0)),
            pl.BlockSpec((n2, hz), lambda i: (0, 0)),
        ],
        out_specs=pl.BlockSpec((bm, n2), lambda i: (i, 0)),
        out_shape=jax.ShapeDtypeStruct((n1, n2), F32),
    )(z1, z2)


def kernel(X1, X2, adj, W_base1, W_mean1, W_logstd1, W_base2, W_mean2,
           W_logstd2, noise1, noise2):
    n1, n2 = adj.shape
    bm = max(n1 // 50, 1)

    xw1 = _xw(X1, W_base1, bm)
    xw2 = _xw(X2, W_base2, bm)
    h1, h2 = _propagate_in(adj, xw1, xw2, bm)
    z1, z2 = _propagate_out(adj, h1, h2, W_mean1, W_logstd1, W_mean2,
                            W_logstd2, noise1, noise2, bm)
    a_pred = _decode(z1, z2, bm)
    return (a_pred, z1, z2)


# speedup vs baseline: 1.0373x; 1.0373x over previous
import functools

import jax
import jax.numpy as jnp
from jax.experimental import pallas as pl
from jax.experimental.pallas import tpu as pltpu

F32 = jnp.float32
BF16 = jnp.bfloat16

_NN = (((1,), (0,)), ((), ()))
_TT = (((0,), (0,)), ((), ()))


def _hi_lo_concat(x, axis=1):
    hi = x.astype(BF16)
    lo = (x - hi.astype(F32)).astype(BF16)
    return jnp.concatenate([hi, lo], axis=axis)


def _sum_halves(y, axis=1):
    w = y.shape[axis] // 2
    if axis == 0:
        return y[:w] + y[w:]
    return y[:, :w] + y[:, w:]


def _xw_body(x_ref, w_ref, o_ref):
    xw = jnp.dot(x_ref[...], w_ref[...], preferred_element_type=F32)
    o_ref[...] = _hi_lo_concat(xw)


def _xw(x, w, bm):
    n, k = x.shape
    h = w.shape[1]
    return pl.pallas_call(
        _xw_body,
        grid=(n // bm,),
        in_specs=[
            pl.BlockSpec((bm, k), lambda i: (i, 0)),
            pl.BlockSpec((k, h), lambda i: (0, 0)),
        ],
        out_specs=pl.BlockSpec((bm, 2 * h), lambda i: (i, 0)),
        out_shape=jax.ShapeDtypeStruct((n, 2 * h), BF16),
    )(x, w)


def _h_body(adj_ref, xw1_ref, xw2_ref, h1_ref, h2_ref, acc1, *, ni):
    i = pl.program_id(0)
    t = adj_ref[...].astype(BF16)
    h2 = jnp.maximum(
        _sum_halves(jax.lax.dot_general(t, xw2_ref[...], _NN,
                                        preferred_element_type=F32)), 0.0)
    h2_ref[...] = _hi_lo_concat(h2)
    c1t = jax.lax.dot_general(xw1_ref[...], t, _TT,
                              preferred_element_type=F32)

    @pl.when(i == 0)
    def _():
        acc1[...] = c1t

    @pl.when(i != 0)
    def _():
        acc1[...] += c1t

    @pl.when(i == ni - 1)
    def _():
        h1t = jnp.maximum(_sum_halves(acc1[...], axis=0), 0.0)
        h1_ref[...] = _hi_lo_concat(h1t.T)


def _propagate_in(adj, xw1, xw2, b):
    n1, n2 = adj.shape
    h2w = xw1.shape[1]
    ni = n1 // b
    return pl.pallas_call(
        functools.partial(_h_body, ni=ni),
        grid=(ni,),
        in_specs=[
            pl.BlockSpec((b, n2), lambda i: (i, 0)),
            pl.BlockSpec((b, h2w), lambda i: (i, 0)),
            pl.BlockSpec((n2, h2w), lambda i: (0, 0)),
        ],
        out_specs=[
            pl.BlockSpec((n2, h2w), lambda i: (0, 0)),
            pl.BlockSpec((b, h2w), lambda i: (i, 0)),
        ],
        out_shape=[
            jax.ShapeDtypeStruct((n2, h2w), BF16),
            jax.ShapeDtypeStruct((n1, h2w), BF16),
        ],
        scratch_shapes=[pltpu.VMEM((h2w, n2), F32)],
    )(adj, xw1, xw2)


def _z_body(adj_ref, h1_ref, h2_ref, wm1_ref, wl1_ref, wm2_ref, wl2_ref,
            n1_ref, n2_ref, z1_ref, z2_ref, acc2, *, ni):
    i = pl.program_id(0)
    t = adj_ref[...].astype(BF16)
    ah1 = _sum_halves(jax.lax.dot_general(t, h1_ref[...], _NN,
                                          preferred_element_type=F32))
    mean1 = jnp.dot(ah1, wm1_ref[...], preferred_element_type=F32)
    logstd1 = jnp.dot(ah1, wl1_ref[...], preferred_element_type=F32)
    z1_ref[...] = n1_ref[...] * jnp.exp(logstd1) + mean1

    c2t = jax.lax.dot_general(h2_ref[...], t, _TT,
                              preferred_element_type=F32)

    @pl.when(i == 0)
    def _():
        acc2[...] = c2t

    @pl.when(i != 0)
    def _():
        acc2[...] += c2t

    @pl.when(i == ni - 1)
    def _():
        ah2t = _sum_halves(acc2[...], axis=0)
        mean2t = jax.lax.dot_general(wm2_ref[...], ah2t, _TT,
                                     preferred_element_type=F32)
        logstd2t = jax.lax.dot_general(wl2_ref[...], ah2t, _TT,
                                       preferred_element_type=F32)
        z2_ref[...] = (n2_ref[...].T * jnp.exp(logstd2t) + mean2t).T


def _propagate_out(adj, h1, h2, wm1, wl1, wm2, wl2, noise1, noise2, b):
    n1, n2 = adj.shape
    h2w = h1.shape[1]
    hz = wm1.shape[1]
    ni = n1 // b
    full = lambda a: pl.BlockSpec(a.shape, lambda i: tuple(0 for _ in a.shape))
    return pl.pallas_call(
        functools.partial(_z_body, ni=ni),
        grid=(ni,),
        in_specs=[
            pl.BlockSpec((b, n2), lambda i: (i, 0)),
            full(h1),
            pl.BlockSpec((b, h2w), lambda i: (i, 0)),
            full(wm1), full(wl1), full(wm2), full(wl2),
            pl.BlockSpec((b, hz), lambda i: (i, 0)),
            full(noise2),
        ],
        out_specs=[
            pl.BlockSpec((b, hz), lambda i: (i, 0)),
            pl.BlockSpec((n2, hz), lambda i: (0, 0)),
        ],
        out_shape=[
            jax.ShapeDtypeStruct((n1, hz), F32),
            jax.ShapeDtypeStruct((n2, hz), F32),
        ],
        scratch_shapes=[pltpu.VMEM((h2w, n2), F32)],
    )(adj, h1, h2, wm1, wl1, wm2, wl2, noise1, noise2)


def _dec_body(z1_ref, z2_ref, a_ref):
    logits = jax.lax.dot_general(z1_ref[...], z2_ref[...],
                                 (((1,), (1,)), ((), ())),
                                 preferred_element_type=F32)
    a_ref[...] = jax.nn.sigmoid(logits)


def _decode(z1, z2, bm):
    n1, hz = z1.shape
    n2 = z2.shape[0]
    return pl.pallas_call(
        _dec_body,
        grid=(n1 // bm,),
        in_specs=[
            pl.BlockSpec((bm, hz), lambda i: (i, 0)),
            pl.BlockSpec((n2, hz), lambda i: (0, 0)),
        ],
        out_specs=pl.BlockSpec((bm, n2), lambda i: (i, 0)),
        out_shape=jax.ShapeDtypeStruct((n1, n2), F32),
    )(z1, z2)


def kernel(X1, X2, adj, W_base1, W_mean1, W_logstd1, W_base2, W_mean2,
           W_logstd2, noise1, noise2):
    n1, n2 = adj.shape
    bm = max(n1 // 25, 1)

    xw1 = _xw(X1, W_base1, bm)
    xw2 = _xw(X2, W_base2, bm)
    h1, h2 = _propagate_in(adj, xw1, xw2, bm)
    z1, z2 = _propagate_out(adj, h1, h2, W_mean1, W_logstd1, W_mean2,
                            W_logstd2, noise1, noise2, bm)
    a_pred = _decode(z1, z2, bm)
    return (a_pred, z1, z2)
